# baseline (speedup 1.0000x reference)
import functools

import jax
import jax.numpy as jnp
import numpy as np
from jax import lax
from jax.experimental import pallas as pl
from jax.experimental.pallas import tpu as pltpu
from jax.experimental.pallas import tpu_sc as plsc

N = 50000
NP = 50176
E = 800000
EP = 819200
D = 16
NW = 32
W_EDGES = EP // NW
CHUNK = 512
NCHUNK = W_EDGES // CHUNK
NJ = CHUNK // 128
RPS = NP // 16
EROWS = EP // 128

_ETS = [("arrival", "transition"), ("waiting", "transition"),
        ("resources", "transition"), ("busy", "transition"),
        ("transition", "arrival"), ("transition", "waiting"),
        ("transition", "resources"), ("transition", "busy")]
_ET_KEYS = ["arrival__to__transition", "waiting__to__transition",
            "resources__to__transition", "busy__to__transition",
            "transition__rev__arrival", "transition__rev__waiting",
            "transition__rev__resources", "transition__rev__busy"]
_NTS = ["arrival", "waiting", "resources", "busy", "transition"]


_sc_mesh = plsc.VectorSubcoreMesh(core_axis_name="c", subcore_axis_name="s")


def _make_sc_multi(n_et):
    @functools.partial(
        pl.kernel,
        out_type=[jax.ShapeDtypeStruct((n_et * 2 * NP, D), jnp.float32),
                  jax.ShapeDtypeStruct((n_et * 2 * NP,), jnp.float32)],
        mesh=_sc_mesh,
        compiler_params=pltpu.CompilerParams(needs_layout_passes=False,
                                             use_tc_tiling_on_sc=False),
        scratch_types=[
            pltpu.VMEM((NJ, 128), jnp.int32),
            pltpu.VMEM((NJ, 128), jnp.int32),
            pltpu.VMEM((NJ, 128), jnp.int32),
            pltpu.VMEM((CHUNK, D), jnp.float32),
            pltpu.VMEM((NJ, 128), jnp.int32),
            pltpu.VMEM((NJ, 128), jnp.int32),
            pltpu.VMEM((NJ, 128), jnp.int32),
            pltpu.VMEM((CHUNK, D), jnp.float32),
            pltpu.VMEM((CHUNK,), jnp.float32),
            pltpu.VMEM((CHUNK,), jnp.float32),
            pltpu.VMEM((CHUNK,), jnp.float32),
            pltpu.VMEM_SHARED((NP,), jnp.float32),
            pltpu.VMEM_SHARED((NP,), jnp.float32),
            pltpu.VMEM_SHARED((NP, D), jnp.float32),
            pltpu.VMEM_SHARED((NP,), jnp.float32),
            pltpu.SemaphoreType.DMA,
            pltpu.SemaphoreType.DMA,
            pltpu.SemaphoreType.DMA,
            pltpu.SemaphoreType.DMA,
            pltpu.SemaphoreType.DMA,
        ],
    )
    def _sc_multi(es_hbm, ed_hbm, x_hbm, asrc_hbm, adst_hbm,
                  num_out, den_out,
                  esq0, edq0, esx0, rows0, esq1, edq1, esx1, rows1,
                  asv, adv, s_v,
                  asrc_sh, adst_sh, num_sh, den_sh,
                  sem_g0, sem_g1, sem_a, sem_s, sem_d):
        cid = lax.axis_index("c")
        sid = lax.axis_index("s")
        wid = cid * 16 + sid
        splat_idx = [jnp.full((16,), t, jnp.int32) for t in range(16)]
        zero16 = jnp.zeros((16,), jnp.float32)

        def _load_idx(esq, edq, esx, et, ch):
            base_row = et * EROWS + wid * (W_EDGES // 128) + ch * NJ
            pltpu.sync_copy(es_hbm.at[pl.ds(base_row, NJ)], esq)
            pltpu.sync_copy(ed_hbm.at[pl.ds(base_row, NJ)], edq)
            off = (et * NP).astype(jnp.int32)
            for j in range(NJ):
                for t in range(0, 128, 16):
                    esx[j, pl.ds(t, 16)] = esq[j, pl.ds(t, 16)] + off

        def _rows_gather(esx, rows_v, sem):
            return [pltpu.async_copy(x_hbm.at[esx.at[j]],
                                     rows_v.at[pl.ds(j * 128, 128)], sem)
                    for j in range(NJ)]

        def _half(esq, edq, rows_v, rdescs):
            ad = []
            for j in range(NJ):
                ad.append(pltpu.async_copy(asrc_sh.at[esq.at[j]],
                                           asv.at[pl.ds(j * 128, 128)],
                                           sem_a))
                ad.append(pltpu.async_copy(adst_sh.at[edq.at[j]],
                                           adv.at[pl.ds(j * 128, 128)],
                                           sem_a))
            for dsc in ad:
                dsc.wait()
            for dsc in rdescs:
                dsc.wait()

            @plsc.parallel_loop(0, CHUNK // 16, 1, unroll=2)
            def _grp(g):
                a_s = asv[pl.ds(g * 16, 16)]
                a_d = adv[pl.ds(g * 16, 16)]
                al = a_s + a_d
                al = jnp.where(al > 0, al, al * jnp.float32(0.2))
                s16 = jnp.exp(al)
                s_v[pl.ds(g * 16, 16)] = s16
                for t in range(16):
                    e = g * 16 + t
                    spl = jnp.take_along_axis(s16, splat_idx[t], axis=0)
                    rows_v[e, :] = rows_v[e, :] * spl

            sd = []
            for j in range(NJ):
                sd.append(pltpu.async_copy(rows_v.at[pl.ds(j * 128, 128)],
                                           num_sh.at[edq.at[j]], sem_s,
                                           add=True))
                sd.append(pltpu.async_copy(s_v.at[pl.ds(j * 128, 128)],
                                           den_sh.at[edq.at[j]], sem_d,
                                           add=True))
            for dsc in sd:
                dsc.wait()

        def _per_et(et, carry):
            pltpu.sync_copy(asrc_hbm.at[pl.ds(et * NP + sid * RPS, RPS)],
                            asrc_sh.at[pl.ds(sid * RPS, RPS)])
            pltpu.sync_copy(adst_hbm.at[pl.ds(et * NP + sid * RPS, RPS)],
                            adst_sh.at[pl.ds(sid * RPS, RPS)])

            def _zr(i, c2):
                rows0[i, :] = zero16
                return c2

            lax.fori_loop(0, CHUNK, _zr, 0)

            def _zs(i, c2):
                s_v[pl.ds(i * 16, 16)] = zero16
                return c2

            lax.fori_loop(0, CHUNK // 16, _zs, 0)
            for k in range(RPS // CHUNK):
                pltpu.sync_copy(
                    rows0, num_sh.at[pl.ds(sid * RPS + k * CHUNK, CHUNK)])
                pltpu.sync_copy(
                    s_v, den_sh.at[pl.ds(sid * RPS + k * CHUNK, CHUNK)])
            _rem = RPS % CHUNK
            if _rem:
                _o = sid * RPS + (RPS // CHUNK) * CHUNK
                pltpu.sync_copy(rows0.at[pl.ds(0, _rem)],
                                num_sh.at[pl.ds(_o, _rem)])
                pltpu.sync_copy(s_v.at[pl.ds(0, _rem)],
                                den_sh.at[pl.ds(_o, _rem)])
            plsc.subcore_barrier()

            def _pair(i, c2):
                _load_idx(esq0, edq0, esx0, et, 2 * i)
                rd0 = _rows_gather(esx0, rows0, sem_g0)
                _load_idx(esq1, edq1, esx1, et, 2 * i + 1)
                rd1 = _rows_gather(esx1, rows1, sem_g1)
                _half(esq0, edq0, rows0, rd0)
                _half(esq1, edq1, rows1, rd1)
                return c2

            lax.fori_loop(0, NCHUNK // 2, _pair, 0)
            plsc.subcore_barrier()

            off = et * 2 * NP + cid * NP + sid * RPS
            pltpu.sync_copy(num_sh.at[pl.ds(sid * RPS, RPS)],
                            num_out.at[pl.ds(off, RPS)])
            pltpu.sync_copy(den_sh.at[pl.ds(sid * RPS, RPS)],
                            den_out.at[pl.ds(off, RPS)])
            return carry

        lax.fori_loop(0, n_et, _per_et, 0)

    return _sc_multi


_sc_multi_8 = _make_sc_multi(8)
_sc_multi_4 = _make_sc_multi(4)


_PAD = NP - N
_BLK = 1792
_NBLK = NP // _BLK


def _row_valid(blk_idx, blk, n_valid):
    rid = blk_idx * blk + lax.broadcasted_iota(jnp.int32, (blk, 1), 0)
    return rid < n_valid


def _prologue_body(xa, xw, xr, xb, xt, wa, ww, wr, wb, wt, pb, ls, ld,
                   ha, hw, hr, hb, ht, asrc, adst):
    i = pl.program_id(0)
    valid = _row_valid(i, _BLK, N)

    def proj(x, w, b):
        h = jnp.dot(x[:], w[:].T, preferred_element_type=jnp.float32) + b
        return jnp.where(valid, h, 0.0)

    hs = {
        "arrival": proj(xa, wa, pb[0]),
        "waiting": proj(xw, ww, pb[1]),
        "resources": proj(xr, wr, pb[2]),
        "busy": proj(xb, wb, pb[3]),
        "transition": proj(xt, wt, pb[4]),
    }
    ha[:] = hs["arrival"]
    hw[:] = hs["waiting"]
    hr[:] = hs["resources"]
    hb[:] = hs["busy"]
    ht[:] = hs["transition"]
    asrc[:] = jnp.stack([(hs[_ETS[j][0]] * ls[j]).sum(-1) for j in range(8)])
    adst[:] = jnp.stack([(hs[_ETS[j][1]] * ld[j]).sum(-1) for j in range(8)])


def _prologue(x_arrival, x_waiting, x_resources, x_busy, x_transition, p1):
    pb = jnp.stack([p1["proj_b"][nt] for nt in _NTS])
    ls = jnp.stack([p1["lin_src"][k][0, 0] for k in _ET_KEYS])
    ld = jnp.stack([p1["lin_dst"][k][0, 0] for k in _ET_KEYS])
    full = lambda *shape: pl.BlockSpec(shape, lambda i: tuple(0 for _ in shape))
    xspec = lambda d: pl.BlockSpec((_BLK, d), lambda i: (i, 0))
    hspec = pl.BlockSpec((_BLK, 16), lambda i: (i, 0))
    aspec = pl.BlockSpec((8, _BLK), lambda i: (0, i))
    out_shape = ([jax.ShapeDtypeStruct((NP, 16), jnp.float32)] * 5
                 + [jax.ShapeDtypeStruct((8, NP), jnp.float32)] * 2)
    in_specs = [xspec(2), xspec(2), xspec(4), xspec(5), xspec(1),
                full(16, 2), full(16, 2), full(16, 4), full(16, 5),
                full(16, 1), full(5, 16), full(8, 16), full(8, 16)]
    out_specs = [hspec] * 5 + [aspec] * 2
    return pl.pallas_call(
        _prologue_body, grid=(_NBLK,), in_specs=in_specs,
        out_specs=out_specs, out_shape=out_shape)(
        x_arrival, x_waiting, x_resources, x_busy, x_transition,
        p1["proj_w"]["arrival"], p1["proj_w"]["waiting"],
        p1["proj_w"]["resources"], p1["proj_w"]["busy"],
        p1["proj_w"]["transition"], pb, ls, ld)


def _c1_body(n0, n1, n2, n3, n4, n5, n6, n7,
             d0, d1, d2, d3, d4, d5, d6, d7,
             kw1, kb1, w2s, b2s, ls2,
             s0, s1, s2, s3, hp0, hp1, hp2, hp3, asrc2, kxsum):
    i = pl.program_id(0)
    valid = _row_valid(i, _BLK, N)

    nums = [n0, n1, n2, n3, n4, n5, n6, n7]
    dens = [d0, d1, d2, d3, d4, d5, d6, d7]
    stks = [s0, s1, s2, s3]
    hps = [hp0, hp1, hp2, hp3]

    @pl.when(i == 0)
    def _init():
        kxsum[:] = jnp.zeros((4, 16), jnp.float32)

    acc = []
    for j in range(4):
        nm = nums[j][:]
        dn = dens[j][:]
        agg = jax.nn.relu((nm[0] + nm[1]) / (dn[0] + dn[1] + 1e-16)[:, None])
        agg = jnp.where(valid, agg, 0.0)
        stks[j][:] = agg
        kx = jnp.tanh(jnp.dot(agg, kw1[:].T,
                              preferred_element_type=jnp.float32) + kb1[:])
        kx = jnp.where(valid, kx, 0.0)
        acc.append(kx.sum(axis=0))
    kxsum[:] = kxsum[:] + jnp.stack(acc)

    a2 = []
    for j in range(4):
        nm = nums[4 + j][:]
        dn = dens[4 + j][:]
        agg = jax.nn.relu((nm[0] + nm[1]) / (dn[0] + dn[1] + 1e-16)[:, None])
        agg = jnp.where(valid, agg, 0.0)
        h2p = jnp.dot(agg, w2s[j].T, preferred_element_type=jnp.float32) + b2s[j]
        h2p = jnp.where(valid, h2p, 0.0)
        hps[j][:] = jnp.concatenate(
            [h2p, jnp.zeros((_BLK, 8), jnp.float32)], axis=1)
        a2.append((h2p * ls2[j]).sum(-1))
    asrc2[:] = jnp.stack(a2)


def _c1(nums, dens, p1, p2):
    kw1 = p1["k_lin_w"]
    kb1 = p1["k_lin_b"]
    w2s = jnp.stack([p2["proj_w"][nt] for nt in _NTS[:4]])
    b2s = jnp.stack([p2["proj_b"][nt] for nt in _NTS[:4]])
    ls2 = jnp.stack([p2["lin_src"][k][0, 0] for k in _ET_KEYS[:4]])
    grid = (_NBLK,)
    nspec = pl.BlockSpec((2, _BLK, 16), lambda i: (0, i, 0))
    dspec = pl.BlockSpec((2, _BLK), lambda i: (0, i))
    full = lambda *shape: pl.BlockSpec(shape, lambda i: tuple(0 for _ in shape))
    ospec16 = pl.BlockSpec((_BLK, 16), lambda i: (i, 0))
    out_shape = ([jax.ShapeDtypeStruct((NP, 16), jnp.float32)] * 8
                 + [jax.ShapeDtypeStruct((4, NP), jnp.float32),
                    jax.ShapeDtypeStruct((4, 16), jnp.float32)])
    out_specs = ([ospec16] * 8
                 + [pl.BlockSpec((4, _BLK), lambda i: (0, i)),
                    full(4, 16)])
    in_specs = ([nspec] * 8 + [dspec] * 8
                + [full(16, 16), full(16,), full(4, 8, 16), full(4, 8),
                   full(4, 8)])
    return pl.pallas_call(
        _c1_body, grid=grid, in_specs=in_specs, out_specs=out_specs,
        out_shape=out_shape)(*nums, *dens, kw1, kb1, w2s, b2s, ls2)


def _c2_body(s0, s1, s2, s3, kxsum, q1, w2t, b2t, ld2, hpt, adst2):
    score = (q1[:][None, :] * (kxsum[:] / N)).sum(-1)
    w = jax.nn.softmax(score)
    res = (w[0] * s0[:] + w[1] * s1[:] + w[2] * s2[:] + w[3] * s3[:])
    htr = jax.nn.relu(res)
    h2p = jnp.dot(htr, w2t[:].T, preferred_element_type=jnp.float32) + b2t[:]
    i = pl.program_id(0)
    valid = _row_valid(i, _BLK, N)
    h2p = jnp.where(valid, h2p, 0.0)
    hpt[:] = jnp.concatenate([h2p, jnp.zeros((_BLK, 8), jnp.float32)], axis=1)
    adst2[:] = jnp.stack([(h2p * ld2[j]).sum(-1) for j in range(4)])


def _c2(stks, kxsum, p1, p2):
    q1 = p1["q"]
    w2t = p2["proj_w"]["transition"]
    b2t = p2["proj_b"]["transition"]
    ld2 = jnp.stack([p2["lin_dst"][k][0, 0] for k in _ET_KEYS[:4]])
    full = lambda *shape: pl.BlockSpec(shape, lambda i: tuple(0 for _ in shape))
    bspec = pl.BlockSpec((_BLK, 16), lambda i: (i, 0))
    out_shape = [jax.ShapeDtypeStruct((NP, 16), jnp.float32),
                 jax.ShapeDtypeStruct((4, NP), jnp.float32)]
    out_specs = [bspec, pl.BlockSpec((4, _BLK), lambda i: (0, i))]
    in_specs = ([bspec] * 4 + [full(4, 16), full(16,), full(8, 16),
                               full(8,), full(4, 8)])
    return pl.pallas_call(
        _c2_body, grid=(_NBLK,), in_specs=in_specs, out_specs=out_specs,
        out_shape=out_shape)(*stks, kxsum, q1, w2t, b2t, ld2)


def _e1_body(n0, n1, n2, n3, d0, d1, d2, d3, kw2, kb2,
             a0, a1, a2, a3, kxsum):
    i = pl.program_id(0)
    valid = _row_valid(i, _BLK, N)
    nums = [n0, n1, n2, n3]
    dens = [d0, d1, d2, d3]
    aggs = [a0, a1, a2, a3]

    @pl.when(i == 0)
    def _init():
        kxsum[:] = jnp.zeros((4, 8), jnp.float32)

    acc = []
    for j in range(4):
        nm = nums[j][:]
        dn = dens[j][:]
        agg = jax.nn.relu((nm[0, :, :8] + nm[1, :, :8])
                          / (dn[0] + dn[1] + 1e-16)[:, None])
        agg = jnp.where(valid, agg, 0.0)
        aggs[j][:] = agg
        kx = jnp.tanh(jnp.dot(agg, kw2[:].T,
                              preferred_element_type=jnp.float32) + kb2[:])
        kx = jnp.where(valid, kx, 0.0)
        acc.append(kx.sum(axis=0))
    kxsum[:] = kxsum[:] + jnp.stack(acc)


def _e1(nums, dens, p2):
    grid = (_NBLK,)
    nspec = pl.BlockSpec((2, _BLK, 16), lambda i: (0, i, 0))
    dspec = pl.BlockSpec((2, _BLK), lambda i: (0, i))
    full = lambda *shape: pl.BlockSpec(shape, lambda i: tuple(0 for _ in shape))
    ospec = pl.BlockSpec((_BLK, 8), lambda i: (i, 0))
    out_shape = ([jax.ShapeDtypeStruct((NP, 8), jnp.float32)] * 4
                 + [jax.ShapeDtypeStruct((4, 8), jnp.float32)])
    out_specs = [ospec] * 4 + [full(4, 8)]
    in_specs = [nspec] * 4 + [dspec] * 4 + [full(8, 8), full(8,)]
    return pl.pallas_call(
        _e1_body, grid=grid, in_specs=in_specs, out_specs=out_specs,
        out_shape=out_shape)(*nums, *dens, p2["k_lin_w"], p2["k_lin_b"])


def _e2_body(a0, a1, a2, a3, kxsum, q2, res, colmax):
    score = (q2[:][None, :] * (kxsum[:] / N)).sum(-1)
    w = jax.nn.softmax(score)
    r = w[0] * a0[:] + w[1] * a1[:] + w[2] * a2[:] + w[3] * a3[:]
    i = pl.program_id(0)
    valid = _row_valid(i, _BLK, N)
    res[:] = r
    rm = jnp.where(valid, r, -jnp.inf).max(axis=0, keepdims=True)

    @pl.when(i == 0)
    def _init():
        colmax[:] = jnp.full((1, 8), -jnp.inf, jnp.float32)

    colmax[:] = jnp.maximum(colmax[:], rm)


def _e2(aggs, kxsum, p2):
    full = lambda *shape: pl.BlockSpec(shape, lambda i: tuple(0 for _ in shape))
    bspec = pl.BlockSpec((_BLK, 8), lambda i: (i, 0))
    out_shape = [jax.ShapeDtypeStruct((NP, 8), jnp.float32),
                 jax.ShapeDtypeStruct((1, 8), jnp.float32)]
    out_specs = [bspec, full(1, 8)]
    in_specs = [bspec] * 4 + [full(4, 8), full(8,)]
    return pl.pallas_call(
        _e2_body, grid=(_NBLK,), in_specs=in_specs, out_specs=out_specs,
        out_shape=out_shape)(*aggs, kxsum, p2["q"])


def _e3_body(res, colmax, ex, colsum):
    i = pl.program_id(0)
    valid = _row_valid(i, _BLK, N)
    e = jnp.exp(res[:] - colmax[:])
    e = jnp.where(valid, e, 0.0)
    ex[:] = e

    @pl.when(i == 0)
    def _init():
        colsum[:] = jnp.zeros((1, 8), jnp.float32)

    colsum[:] = colsum[:] + e.sum(axis=0, keepdims=True)


def _e3(res, colmax):
    full = lambda *shape: pl.BlockSpec(shape, lambda i: tuple(0 for _ in shape))
    bspec = pl.BlockSpec((_BLK, 8), lambda i: (i, 0))
    out_shape = [jax.ShapeDtypeStruct((NP, 8), jnp.float32),
                 jax.ShapeDtypeStruct((1, 8), jnp.float32)]
    return pl.pallas_call(
        _e3_body, grid=(_NBLK,), in_specs=[bspec, full(1, 8)],
        out_specs=[bspec, full(1, 8)],
        out_shape=out_shape)(res, colmax)


def _e4_body(ex, colsum, out):
    out[:] = ex[:] / colsum[:]


def _e4(ex, colsum):
    bspec = pl.BlockSpec((_BLK, 8), lambda i: (i, 0))
    full = lambda *shape: pl.BlockSpec(shape, lambda i: tuple(0 for _ in shape))
    return pl.pallas_call(
        _e4_body, grid=(_NBLK,), in_specs=[bspec, full(1, 8)],
        out_specs=bspec,
        out_shape=jax.ShapeDtypeStruct((NP, 8), jnp.float32))(ex, colsum)


def _pad_edges(ei):
    es = jnp.concatenate([ei[0].astype(jnp.int32),
                          jnp.zeros((EP - E,), jnp.int32)])
    ed = jnp.concatenate([ei[1].astype(jnp.int32),
                          jnp.full((EP - E,), NP - 1, jnp.int32)])
    return es.reshape(EROWS, 128), ed.reshape(EROWS, 128)


def kernel(x_arrival, x_waiting, x_resources, x_busy, x_transition,
           ei_a2t, ei_w2t, ei_r2t, ei_b2t, ei_t2a, ei_t2w, ei_t2r, ei_t2b,
           params1, params2):
    p1, p2 = params1, params2
    eis = [ei_a2t, ei_w2t, ei_r2t, ei_b2t, ei_t2a, ei_t2w, ei_t2r, ei_t2b]
    epads = [_pad_edges(ei) for ei in eis]

    ha, hw, hr, hb, ht, asrc1, adst1 = _prologue(
        x_arrival, x_waiting, x_resources, x_busy, x_transition, p1)
    htabs = {"arrival": ha, "waiting": hw, "resources": hr, "busy": hb,
             "transition": ht}

    es_all = jnp.concatenate([epads[j][0] for j in range(8)])
    ed_all = jnp.concatenate([epads[j][1] for j in range(8)])
    x_all = jnp.concatenate([htabs[_ETS[j][0]] for j in range(8)])
    num_all, den_all = _sc_multi_8(es_all, ed_all, x_all,
                                   asrc1.reshape(-1), adst1.reshape(-1))
    num_all = num_all.reshape(8, 2, NP, 16)
    den_all = den_all.reshape(8, 2, NP)
    nums = [num_all[j] for j in range(8)]
    dens = [den_all[j] for j in range(8)]

    c1out = _c1(nums, dens, p1, p2)
    stks = c1out[0:4]
    hps = c1out[4:8]
    asrc2 = c1out[8]
    kxsum1 = c1out[9]

    hpt, adst2 = _c2(stks, kxsum1, p1, p2)

    es_all2 = jnp.concatenate([epads[j][0] for j in range(4)])
    ed_all2 = jnp.concatenate([epads[j][1] for j in range(4)])
    x_all2 = jnp.concatenate(list(hps))
    num_all2, den_all2 = _sc_multi_4(es_all2, ed_all2, x_all2,
                                     asrc2.reshape(-1), adst2.reshape(-1))
    num_all2 = num_all2.reshape(4, 2, NP, 16)
    den_all2 = den_all2.reshape(4, 2, NP)
    nums2 = [num_all2[j] for j in range(4)]
    dens2 = [den_all2[j] for j in range(4)]

    e1out = _e1(nums2, dens2, p2)
    aggs2 = e1out[0:4]
    kxsum2 = e1out[4]
    res, colmax = _e2(aggs2, kxsum2, p2)
    ex, colsum = _e3(res, colmax)
    out = _e4(ex, colsum)
    return out[:N]


# speedup vs baseline: 1.3397x; 1.3397x over previous
import functools

import jax
import jax.numpy as jnp
import numpy as np
from jax import lax
from jax.experimental import pallas as pl
from jax.experimental.pallas import tpu as pltpu
from jax.experimental.pallas import tpu_sc as plsc

N = 50000
NP = 50176
E = 800000
EP = 819200
D = 16
NW = 32
W_EDGES = EP // NW
CHUNK = 512
NCHUNK = W_EDGES // CHUNK
NJ = CHUNK // 128
RPS = NP // 16
EROWS = EP // 128

_ETS = [("arrival", "transition"), ("waiting", "transition"),
        ("resources", "transition"), ("busy", "transition"),
        ("transition", "arrival"), ("transition", "waiting"),
        ("transition", "resources"), ("transition", "busy")]
_ET_KEYS = ["arrival__to__transition", "waiting__to__transition",
            "resources__to__transition", "busy__to__transition",
            "transition__rev__arrival", "transition__rev__waiting",
            "transition__rev__resources", "transition__rev__busy"]
_NTS = ["arrival", "waiting", "resources", "busy", "transition"]


_sc_mesh = plsc.VectorSubcoreMesh(core_axis_name="c", subcore_axis_name="s")


@functools.partial(
    pl.kernel,
    out_type=[jax.ShapeDtypeStruct((2 * NP, D), jnp.float32),
              jax.ShapeDtypeStruct((2 * NP,), jnp.float32)],
    mesh=_sc_mesh,
    compiler_params=pltpu.CompilerParams(needs_layout_passes=False,
                                         use_tc_tiling_on_sc=False),
    scratch_types=[
        pltpu.VMEM((NJ, 128), jnp.int32),
        pltpu.VMEM((NJ, 128), jnp.int32),
        pltpu.VMEM((CHUNK, D), jnp.float32),
        pltpu.VMEM((NJ, 128), jnp.int32),
        pltpu.VMEM((NJ, 128), jnp.int32),
        pltpu.VMEM((CHUNK, D), jnp.float32),
        pltpu.VMEM((CHUNK,), jnp.float32),
        pltpu.VMEM((CHUNK,), jnp.float32),
        pltpu.VMEM((CHUNK,), jnp.float32),
        pltpu.VMEM_SHARED((NP,), jnp.float32),
        pltpu.VMEM_SHARED((NP,), jnp.float32),
        pltpu.VMEM_SHARED((NP, D), jnp.float32),
        pltpu.VMEM_SHARED((NP,), jnp.float32),
        pltpu.SemaphoreType.DMA,
        pltpu.SemaphoreType.DMA,
        pltpu.SemaphoreType.DMA,
        pltpu.SemaphoreType.DMA,
        pltpu.SemaphoreType.DMA,
    ],
)
def _sc_edge_pass(es_hbm, ed_hbm, x_hbm, asrc_hbm, adst_hbm,
                  num_out, den_out,
                  esq0, edq0, rows0, esq1, edq1, rows1,
                  asv, adv, s_v,
                  asrc_sh, adst_sh, num_sh, den_sh,
                  sem_g0, sem_g1, sem_a, sem_s, sem_d):
    cid = lax.axis_index("c")
    sid = lax.axis_index("s")
    wid = cid * 16 + sid

    pltpu.sync_copy(asrc_hbm.at[pl.ds(sid * RPS, RPS)],
                    asrc_sh.at[pl.ds(sid * RPS, RPS)])
    pltpu.sync_copy(adst_hbm.at[pl.ds(sid * RPS, RPS)],
                    adst_sh.at[pl.ds(sid * RPS, RPS)])

    zero16 = jnp.zeros((16,), jnp.float32)

    def _zr(i, carry):
        rows0[i, :] = zero16
        return carry

    lax.fori_loop(0, CHUNK, _zr, 0)

    def _zs(i, carry):
        s_v[pl.ds(i * 16, 16)] = zero16
        return carry

    lax.fori_loop(0, CHUNK // 16, _zs, 0)

    for k in range(RPS // CHUNK):
        pltpu.sync_copy(rows0,
                        num_sh.at[pl.ds(sid * RPS + k * CHUNK, CHUNK)])
        pltpu.sync_copy(s_v,
                        den_sh.at[pl.ds(sid * RPS + k * CHUNK, CHUNK)])
    _rem = RPS % CHUNK
    if _rem:
        _o = sid * RPS + (RPS // CHUNK) * CHUNK
        pltpu.sync_copy(rows0.at[pl.ds(0, _rem)],
                        num_sh.at[pl.ds(_o, _rem)])
        pltpu.sync_copy(s_v.at[pl.ds(0, _rem)],
                        den_sh.at[pl.ds(_o, _rem)])
    plsc.subcore_barrier()

    splat_idx = [jnp.full((16,), t, jnp.int32) for t in range(16)]

    def _load_idx(esq, edq, ch):
        base_row = wid * (W_EDGES // 128) + ch * NJ
        pltpu.sync_copy(es_hbm.at[pl.ds(base_row, NJ)], esq)
        pltpu.sync_copy(ed_hbm.at[pl.ds(base_row, NJ)], edq)

    def _rows_gather(esq, rows_v, sem):
        return [pltpu.async_copy(x_hbm.at[esq.at[j]],
                                 rows_v.at[pl.ds(j * 128, 128)], sem)
                for j in range(NJ)]

    def _half(esq, edq, rows_v, rdescs):
        ad = []
        for j in range(NJ):
            ad.append(pltpu.async_copy(asrc_sh.at[esq.at[j]],
                                       asv.at[pl.ds(j * 128, 128)], sem_a))
            ad.append(pltpu.async_copy(adst_sh.at[edq.at[j]],
                                       adv.at[pl.ds(j * 128, 128)], sem_a))
        for dsc in ad:
            dsc.wait()
        for dsc in rdescs:
            dsc.wait()

        @plsc.parallel_loop(0, CHUNK // 16, 1, unroll=4)
        def _grp(g):
            a_s = asv[pl.ds(g * 16, 16)]
            a_d = adv[pl.ds(g * 16, 16)]
            al = a_s + a_d
            al = jnp.where(al > 0, al, al * jnp.float32(0.2))
            s16 = jnp.exp(al)
            s_v[pl.ds(g * 16, 16)] = s16
            for t in range(16):
                e = g * 16 + t
                spl = jnp.take_along_axis(s16, splat_idx[t], axis=0)
                rows_v[e, :] = rows_v[e, :] * spl

        sd = []
        for j in range(NJ):
            sd.append(pltpu.async_copy(rows_v.at[pl.ds(j * 128, 128)],
                                       num_sh.at[edq.at[j]], sem_s, add=True))
            sd.append(pltpu.async_copy(s_v.at[pl.ds(j * 128, 128)],
                                       den_sh.at[edq.at[j]], sem_d, add=True))
        for dsc in sd:
            dsc.wait()

    def _pair(i, carry):
        _load_idx(esq0, edq0, 2 * i)
        rd0 = _rows_gather(esq0, rows0, sem_g0)
        _load_idx(esq1, edq1, 2 * i + 1)
        rd1 = _rows_gather(esq1, rows1, sem_g1)
        _half(esq0, edq0, rows0, rd0)
        _half(esq1, edq1, rows1, rd1)
        return carry

    lax.fori_loop(0, NCHUNK // 2, _pair, 0)
    plsc.subcore_barrier()

    off = cid * NP + sid * RPS
    pltpu.sync_copy(num_sh.at[pl.ds(sid * RPS, RPS)],
                    num_out.at[pl.ds(off, RPS)])
    pltpu.sync_copy(den_sh.at[pl.ds(sid * RPS, RPS)],
                    den_out.at[pl.ds(off, RPS)])


_PAD = NP - N
_BLK = 1792
_NBLK = NP // _BLK


def _row_valid(blk_idx, blk, n_valid):
    rid = blk_idx * blk + lax.broadcasted_iota(jnp.int32, (blk, 1), 0)
    return rid < n_valid


def _prologue_body(xa, xw, xr, xb, xt, wa, ww, wr, wb, wt, pb, ls, ld,
                   ha, hw, hr, hb, ht, asrc, adst):
    i = pl.program_id(0)
    valid = _row_valid(i, _BLK, N)

    def proj(x, w, b):
        h = jnp.dot(x[:], w[:].T, preferred_element_type=jnp.float32) + b
        return jnp.where(valid, h, 0.0)

    hs = {
        "arrival": proj(xa, wa, pb[0]),
        "waiting": proj(xw, ww, pb[1]),
        "resources": proj(xr, wr, pb[2]),
        "busy": proj(xb, wb, pb[3]),
        "transition": proj(xt, wt, pb[4]),
    }
    ha[:] = hs["arrival"]
    hw[:] = hs["waiting"]
    hr[:] = hs["resources"]
    hb[:] = hs["busy"]
    ht[:] = hs["transition"]
    asrc[:] = jnp.stack([(hs[_ETS[j][0]] * ls[j]).sum(-1) for j in range(8)])
    adst[:] = jnp.stack([(hs[_ETS[j][1]] * ld[j]).sum(-1) for j in range(8)])


def _prologue(x_arrival, x_waiting, x_resources, x_busy, x_transition, p1):
    pb = jnp.stack([p1["proj_b"][nt] for nt in _NTS])
    ls = jnp.stack([p1["lin_src"][k][0, 0] for k in _ET_KEYS])
    ld = jnp.stack([p1["lin_dst"][k][0, 0] for k in _ET_KEYS])
    full = lambda *shape: pl.BlockSpec(shape, lambda i: tuple(0 for _ in shape))
    xspec = lambda d: pl.BlockSpec((_BLK, d), lambda i: (i, 0))
    hspec = pl.BlockSpec((_BLK, 16), lambda i: (i, 0))
    aspec = pl.BlockSpec((8, _BLK), lambda i: (0, i))
    out_shape = ([jax.ShapeDtypeStruct((NP, 16), jnp.float32)] * 5
                 + [jax.ShapeDtypeStruct((8, NP), jnp.float32)] * 2)
    in_specs = [xspec(2), xspec(2), xspec(4), xspec(5), xspec(1),
                full(16, 2), full(16, 2), full(16, 4), full(16, 5),
                full(16, 1), full(5, 16), full(8, 16), full(8, 16)]
    out_specs = [hspec] * 5 + [aspec] * 2
    return pl.pallas_call(
        _prologue_body, grid=(_NBLK,), in_specs=in_specs,
        out_specs=out_specs, out_shape=out_shape)(
        x_arrival, x_waiting, x_resources, x_busy, x_transition,
        p1["proj_w"]["arrival"], p1["proj_w"]["waiting"],
        p1["proj_w"]["resources"], p1["proj_w"]["busy"],
        p1["proj_w"]["transition"], pb, ls, ld)


def _c1_body(n0, n1, n2, n3, n4, n5, n6, n7,
             d0, d1, d2, d3, d4, d5, d6, d7,
             kw1, kb1, w2s, b2s, ls2,
             s0, s1, s2, s3, hp0, hp1, hp2, hp3, asrc2, kxsum):
    i = pl.program_id(0)
    valid = _row_valid(i, _BLK, N)

    nums = [n0, n1, n2, n3, n4, n5, n6, n7]
    dens = [d0, d1, d2, d3, d4, d5, d6, d7]
    stks = [s0, s1, s2, s3]
    hps = [hp0, hp1, hp2, hp3]

    @pl.when(i == 0)
    def _init():
        kxsum[:] = jnp.zeros((4, 16), jnp.float32)

    acc = []
    for j in range(4):
        nm = nums[j][:]
        dn = dens[j][:]
        agg = jax.nn.relu((nm[0] + nm[1]) / (dn[0] + dn[1] + 1e-16)[:, None])
        agg = jnp.where(valid, agg, 0.0)
        stks[j][:] = agg
        kx = jnp.tanh(jnp.dot(agg, kw1[:].T,
                              preferred_element_type=jnp.float32) + kb1[:])
        kx = jnp.where(valid, kx, 0.0)
        acc.append(kx.sum(axis=0))
    kxsum[:] = kxsum[:] + jnp.stack(acc)

    a2 = []
    for j in range(4):
        nm = nums[4 + j][:]
        dn = dens[4 + j][:]
        agg = jax.nn.relu((nm[0] + nm[1]) / (dn[0] + dn[1] + 1e-16)[:, None])
        agg = jnp.where(valid, agg, 0.0)
        h2p = jnp.dot(agg, w2s[j].T, preferred_element_type=jnp.float32) + b2s[j]
        h2p = jnp.where(valid, h2p, 0.0)
        hps[j][:] = jnp.concatenate(
            [h2p, jnp.zeros((_BLK, 8), jnp.float32)], axis=1)
        a2.append((h2p * ls2[j]).sum(-1))
    asrc2[:] = jnp.stack(a2)


def _c1(nums, dens, p1, p2):
    kw1 = p1["k_lin_w"]
    kb1 = p1["k_lin_b"]
    w2s = jnp.stack([p2["proj_w"][nt] for nt in _NTS[:4]])
    b2s = jnp.stack([p2["proj_b"][nt] for nt in _NTS[:4]])
    ls2 = jnp.stack([p2["lin_src"][k][0, 0] for k in _ET_KEYS[:4]])
    grid = (_NBLK,)
    nspec = pl.BlockSpec((2, _BLK, 16), lambda i: (0, i, 0))
    dspec = pl.BlockSpec((2, _BLK), lambda i: (0, i))
    full = lambda *shape: pl.BlockSpec(shape, lambda i: tuple(0 for _ in shape))
    ospec16 = pl.BlockSpec((_BLK, 16), lambda i: (i, 0))
    out_shape = ([jax.ShapeDtypeStruct((NP, 16), jnp.float32)] * 8
                 + [jax.ShapeDtypeStruct((4, NP), jnp.float32),
                    jax.ShapeDtypeStruct((4, 16), jnp.float32)])
    out_specs = ([ospec16] * 8
                 + [pl.BlockSpec((4, _BLK), lambda i: (0, i)),
                    full(4, 16)])
    in_specs = ([nspec] * 8 + [dspec] * 8
                + [full(16, 16), full(16,), full(4, 8, 16), full(4, 8),
                   full(4, 8)])
    return pl.pallas_call(
        _c1_body, grid=grid, in_specs=in_specs, out_specs=out_specs,
        out_shape=out_shape)(*nums, *dens, kw1, kb1, w2s, b2s, ls2)


def _c2_body(s0, s1, s2, s3, kxsum, q1, w2t, b2t, ld2, hpt, adst2):
    score = (q1[:][None, :] * (kxsum[:] / N)).sum(-1)
    w = jax.nn.softmax(score)
    res = (w[0] * s0[:] + w[1] * s1[:] + w[2] * s2[:] + w[3] * s3[:])
    htr = jax.nn.relu(res)
    h2p = jnp.dot(htr, w2t[:].T, preferred_element_type=jnp.float32) + b2t[:]
    i = pl.program_id(0)
    valid = _row_valid(i, _BLK, N)
    h2p = jnp.where(valid, h2p, 0.0)
    hpt[:] = jnp.concatenate([h2p, jnp.zeros((_BLK, 8), jnp.float32)], axis=1)
    adst2[:] = jnp.stack([(h2p * ld2[j]).sum(-1) for j in range(4)])


def _c2(stks, kxsum, p1, p2):
    q1 = p1["q"]
    w2t = p2["proj_w"]["transition"]
    b2t = p2["proj_b"]["transition"]
    ld2 = jnp.stack([p2["lin_dst"][k][0, 0] for k in _ET_KEYS[:4]])
    full = lambda *shape: pl.BlockSpec(shape, lambda i: tuple(0 for _ in shape))
    bspec = pl.BlockSpec((_BLK, 16), lambda i: (i, 0))
    out_shape = [jax.ShapeDtypeStruct((NP, 16), jnp.float32),
                 jax.ShapeDtypeStruct((4, NP), jnp.float32)]
    out_specs = [bspec, pl.BlockSpec((4, _BLK), lambda i: (0, i))]
    in_specs = ([bspec] * 4 + [full(4, 16), full(16,), full(8, 16),
                               full(8,), full(4, 8)])
    return pl.pallas_call(
        _c2_body, grid=(_NBLK,), in_specs=in_specs, out_specs=out_specs,
        out_shape=out_shape)(*stks, kxsum, q1, w2t, b2t, ld2)


def _e1_body(n0, n1, n2, n3, d0, d1, d2, d3, kw2, kb2,
             a0, a1, a2, a3, kxsum):
    i = pl.program_id(0)
    valid = _row_valid(i, _BLK, N)
    nums = [n0, n1, n2, n3]
    dens = [d0, d1, d2, d3]
    aggs = [a0, a1, a2, a3]

    @pl.when(i == 0)
    def _init():
        kxsum[:] = jnp.zeros((4, 8), jnp.float32)

    acc = []
    for j in range(4):
        nm = nums[j][:]
        dn = dens[j][:]
        agg = jax.nn.relu((nm[0, :, :8] + nm[1, :, :8])
                          / (dn[0] + dn[1] + 1e-16)[:, None])
        agg = jnp.where(valid, agg, 0.0)
        aggs[j][:] = agg
        kx = jnp.tanh(jnp.dot(agg, kw2[:].T,
                              preferred_element_type=jnp.float32) + kb2[:])
        kx = jnp.where(valid, kx, 0.0)
        acc.append(kx.sum(axis=0))
    kxsum[:] = kxsum[:] + jnp.stack(acc)


def _e1(nums, dens, p2):
    grid = (_NBLK,)
    nspec = pl.BlockSpec((2, _BLK, 16), lambda i: (0, i, 0))
    dspec = pl.BlockSpec((2, _BLK), lambda i: (0, i))
    full = lambda *shape: pl.BlockSpec(shape, lambda i: tuple(0 for _ in shape))
    ospec = pl.BlockSpec((_BLK, 8), lambda i: (i, 0))
    out_shape = ([jax.ShapeDtypeStruct((NP, 8), jnp.float32)] * 4
                 + [jax.ShapeDtypeStruct((4, 8), jnp.float32)])
    out_specs = [ospec] * 4 + [full(4, 8)]
    in_specs = [nspec] * 4 + [dspec] * 4 + [full(8, 8), full(8,)]
    return pl.pallas_call(
        _e1_body, grid=grid, in_specs=in_specs, out_specs=out_specs,
        out_shape=out_shape)(*nums, *dens, p2["k_lin_w"], p2["k_lin_b"])


def _e2_body(a0, a1, a2, a3, kxsum, q2, res, colmax):
    score = (q2[:][None, :] * (kxsum[:] / N)).sum(-1)
    w = jax.nn.softmax(score)
    r = w[0] * a0[:] + w[1] * a1[:] + w[2] * a2[:] + w[3] * a3[:]
    i = pl.program_id(0)
    valid = _row_valid(i, _BLK, N)
    res[:] = r
    rm = jnp.where(valid, r, -jnp.inf).max(axis=0, keepdims=True)

    @pl.when(i == 0)
    def _init():
        colmax[:] = jnp.full((1, 8), -jnp.inf, jnp.float32)

    colmax[:] = jnp.maximum(colmax[:], rm)


def _e2(aggs, kxsum, p2):
    full = lambda *shape: pl.BlockSpec(shape, lambda i: tuple(0 for _ in shape))
    bspec = pl.BlockSpec((_BLK, 8), lambda i: (i, 0))
    out_shape = [jax.ShapeDtypeStruct((NP, 8), jnp.float32),
                 jax.ShapeDtypeStruct((1, 8), jnp.float32)]
    out_specs = [bspec, full(1, 8)]
    in_specs = [bspec] * 4 + [full(4, 8), full(8,)]
    return pl.pallas_call(
        _e2_body, grid=(_NBLK,), in_specs=in_specs, out_specs=out_specs,
        out_shape=out_shape)(*aggs, kxsum, p2["q"])


def _e3_body(res, colmax, ex, colsum):
    i = pl.program_id(0)
    valid = _row_valid(i, _BLK, N)
    e = jnp.exp(res[:] - colmax[:])
    e = jnp.where(valid, e, 0.0)
    ex[:] = e

    @pl.when(i == 0)
    def _init():
        colsum[:] = jnp.zeros((1, 8), jnp.float32)

    colsum[:] = colsum[:] + e.sum(axis=0, keepdims=True)


def _e3(res, colmax):
    full = lambda *shape: pl.BlockSpec(shape, lambda i: tuple(0 for _ in shape))
    bspec = pl.BlockSpec((_BLK, 8), lambda i: (i, 0))
    out_shape = [jax.ShapeDtypeStruct((NP, 8), jnp.float32),
                 jax.ShapeDtypeStruct((1, 8), jnp.float32)]
    return pl.pallas_call(
        _e3_body, grid=(_NBLK,), in_specs=[bspec, full(1, 8)],
        out_specs=[bspec, full(1, 8)],
        out_shape=out_shape)(res, colmax)


def _e4_body(ex, colsum, out):
    out[:] = ex[:] / colsum[:]


def _e4(ex, colsum):
    bspec = pl.BlockSpec((_BLK, 8), lambda i: (i, 0))
    full = lambda *shape: pl.BlockSpec(shape, lambda i: tuple(0 for _ in shape))
    return pl.pallas_call(
        _e4_body, grid=(_NBLK,), in_specs=[bspec, full(1, 8)],
        out_specs=bspec,
        out_shape=jax.ShapeDtypeStruct((NP, 8), jnp.float32))(ex, colsum)


def _pad_edges(ei):
    es = jnp.concatenate([ei[0].astype(jnp.int32),
                          jnp.zeros((EP - E,), jnp.int32)])
    ed = jnp.concatenate([ei[1].astype(jnp.int32),
                          jnp.full((EP - E,), NP - 1, jnp.int32)])
    return es.reshape(EROWS, 128), ed.reshape(EROWS, 128)


def kernel(x_arrival, x_waiting, x_resources, x_busy, x_transition,
           ei_a2t, ei_w2t, ei_r2t, ei_b2t, ei_t2a, ei_t2w, ei_t2r, ei_t2b,
           params1, params2):
    p1, p2 = params1, params2
    eis = [ei_a2t, ei_w2t, ei_r2t, ei_b2t, ei_t2a, ei_t2w, ei_t2r, ei_t2b]
    epads = [_pad_edges(ei) for ei in eis]

    ha, hw, hr, hb, ht, asrc1, adst1 = _prologue(
        x_arrival, x_waiting, x_resources, x_busy, x_transition, p1)
    htabs = {"arrival": ha, "waiting": hw, "resources": hr, "busy": hb,
             "transition": ht}

    nums, dens = [], []
    for j in range(8):
        es, ed = epads[j]
        num, den = _sc_edge_pass(es, ed, htabs[_ETS[j][0]],
                                 asrc1[j], adst1[j])
        nums.append(num.reshape(2, NP, 16))
        dens.append(den.reshape(2, NP))

    c1out = _c1(nums, dens, p1, p2)
    stks = c1out[0:4]
    hps = c1out[4:8]
    asrc2 = c1out[8]
    kxsum1 = c1out[9]

    hpt, adst2 = _c2(stks, kxsum1, p1, p2)

    nums2, dens2 = [], []
    for j in range(4):
        es, ed = epads[j]
        num, den = _sc_edge_pass(es, ed, hps[j], asrc2[j], adst2[j])
        nums2.append(num.reshape(2, NP, 16))
        dens2.append(den.reshape(2, NP))

    e1out = _e1(nums2, dens2, p2)
    aggs2 = e1out[0:4]
    kxsum2 = e1out[4]
    res, colmax = _e2(aggs2, kxsum2, p2)
    ex, colsum = _e3(res, colmax)
    out = _e4(ex, colsum)
    return out[:N]


# speedup vs baseline: 1.4922x; 1.1138x over previous
import functools

import jax
import jax.numpy as jnp
import numpy as np
from jax import lax
from jax.experimental import pallas as pl
from jax.experimental.pallas import tpu as pltpu
from jax.experimental.pallas import tpu_sc as plsc

N = 50000
NP = 50176
E = 800000
EP = 819200
D = 16
NW = 32
W_EDGES = EP // NW
CHUNK = 1280
NCHUNK = W_EDGES // CHUNK
NJ = CHUNK // 128
RPS = NP // 16
EROWS = EP // 128

_ETS = [("arrival", "transition"), ("waiting", "transition"),
        ("resources", "transition"), ("busy", "transition"),
        ("transition", "arrival"), ("transition", "waiting"),
        ("transition", "resources"), ("transition", "busy")]
_ET_KEYS = ["arrival__to__transition", "waiting__to__transition",
            "resources__to__transition", "busy__to__transition",
            "transition__rev__arrival", "transition__rev__waiting",
            "transition__rev__resources", "transition__rev__busy"]
_NTS = ["arrival", "waiting", "resources", "busy", "transition"]


_sc_mesh = plsc.VectorSubcoreMesh(core_axis_name="c", subcore_axis_name="s")


@functools.partial(
    pl.kernel,
    out_type=[jax.ShapeDtypeStruct((2 * NP, D), jnp.float32),
              jax.ShapeDtypeStruct((2 * NP,), jnp.float32)],
    mesh=_sc_mesh,
    compiler_params=pltpu.CompilerParams(needs_layout_passes=False,
                                         use_tc_tiling_on_sc=False),
    scratch_types=[
        pltpu.VMEM((NJ, 128), jnp.int32),
        pltpu.VMEM((NJ, 128), jnp.int32),
        pltpu.VMEM((CHUNK, D), jnp.float32),
        pltpu.VMEM((NJ, 128), jnp.int32),
        pltpu.VMEM((NJ, 128), jnp.int32),
        pltpu.VMEM((CHUNK, D), jnp.float32),
        pltpu.VMEM((CHUNK,), jnp.float32),
        pltpu.VMEM((CHUNK,), jnp.float32),
        pltpu.VMEM((CHUNK,), jnp.float32),
        pltpu.VMEM_SHARED((NP,), jnp.float32),
        pltpu.VMEM_SHARED((NP,), jnp.float32),
        pltpu.VMEM_SHARED((NP, D), jnp.float32),
        pltpu.VMEM_SHARED((NP,), jnp.float32),
        pltpu.SemaphoreType.DMA,
        pltpu.SemaphoreType.DMA,
        pltpu.SemaphoreType.DMA,
        pltpu.SemaphoreType.DMA,
        pltpu.SemaphoreType.DMA,
    ],
)
def _sc_edge_pass(es_hbm, ed_hbm, x_hbm, asrc_hbm, adst_hbm,
                  num_out, den_out,
                  esq0, edq0, rows0, esq1, edq1, rows1,
                  asv, adv, s_v,
                  asrc_sh, adst_sh, num_sh, den_sh,
                  sem_g0, sem_g1, sem_a, sem_s, sem_d):
    cid = lax.axis_index("c")
    sid = lax.axis_index("s")
    wid = cid * 16 + sid

    pltpu.sync_copy(asrc_hbm.at[pl.ds(sid * RPS, RPS)],
                    asrc_sh.at[pl.ds(sid * RPS, RPS)])
    pltpu.sync_copy(adst_hbm.at[pl.ds(sid * RPS, RPS)],
                    adst_sh.at[pl.ds(sid * RPS, RPS)])

    zero16 = jnp.zeros((16,), jnp.float32)

    def _zr(i, carry):
        rows0[i, :] = zero16
        return carry

    lax.fori_loop(0, CHUNK, _zr, 0)

    def _zs(i, carry):
        s_v[pl.ds(i * 16, 16)] = zero16
        return carry

    lax.fori_loop(0, CHUNK // 16, _zs, 0)

    for k in range(RPS // CHUNK):
        pltpu.sync_copy(rows0,
                        num_sh.at[pl.ds(sid * RPS + k * CHUNK, CHUNK)])
        pltpu.sync_copy(s_v,
                        den_sh.at[pl.ds(sid * RPS + k * CHUNK, CHUNK)])
    _rem = RPS % CHUNK
    if _rem:
        _o = sid * RPS + (RPS // CHUNK) * CHUNK
        pltpu.sync_copy(rows0.at[pl.ds(0, _rem)],
                        num_sh.at[pl.ds(_o, _rem)])
        pltpu.sync_copy(s_v.at[pl.ds(0, _rem)],
                        den_sh.at[pl.ds(_o, _rem)])
    plsc.subcore_barrier()

    splat_idx = [jnp.full((16,), t, jnp.int32) for t in range(16)]

    def _load_idx(esq, edq, ch):
        base_row = wid * (W_EDGES // 128) + ch * NJ
        pltpu.sync_copy(es_hbm.at[pl.ds(base_row, NJ)], esq)
        pltpu.sync_copy(ed_hbm.at[pl.ds(base_row, NJ)], edq)

    def _rows_gather(esq, rows_v, sem):
        return [pltpu.async_copy(x_hbm.at[esq.at[j]],
                                 rows_v.at[pl.ds(j * 128, 128)], sem)
                for j in range(NJ)]

    def _half(esq, edq, rows_v, rdescs):
        ad = []
        for j in range(NJ):
            ad.append(pltpu.async_copy(asrc_sh.at[esq.at[j]],
                                       asv.at[pl.ds(j * 128, 128)], sem_a))
            ad.append(pltpu.async_copy(adst_sh.at[edq.at[j]],
                                       adv.at[pl.ds(j * 128, 128)], sem_a))
        for dsc in ad:
            dsc.wait()
        for dsc in rdescs:
            dsc.wait()

        @plsc.parallel_loop(0, CHUNK // 16, 1, unroll=2)
        def _grp(g):
            a_s = asv[pl.ds(g * 16, 16)]
            a_d = adv[pl.ds(g * 16, 16)]
            al = a_s + a_d
            al = jnp.where(al > 0, al, al * jnp.float32(0.2))
            s16 = jnp.exp(al)
            s_v[pl.ds(g * 16, 16)] = s16
            for t in range(16):
                e = g * 16 + t
                spl = jnp.take_along_axis(s16, splat_idx[t], axis=0)
                rows_v[e, :] = rows_v[e, :] * spl

        sd = []
        for j in range(NJ):
            sd.append(pltpu.async_copy(rows_v.at[pl.ds(j * 128, 128)],
                                       num_sh.at[edq.at[j]], sem_s, add=True))
            sd.append(pltpu.async_copy(s_v.at[pl.ds(j * 128, 128)],
                                       den_sh.at[edq.at[j]], sem_d, add=True))
        for dsc in sd:
            dsc.wait()

    def _pair(i, carry):
        _load_idx(esq0, edq0, 2 * i)
        rd0 = _rows_gather(esq0, rows0, sem_g0)
        _load_idx(esq1, edq1, 2 * i + 1)
        rd1 = _rows_gather(esq1, rows1, sem_g1)
        _half(esq0, edq0, rows0, rd0)
        _half(esq1, edq1, rows1, rd1)
        return carry

    lax.fori_loop(0, NCHUNK // 2, _pair, 0)
    plsc.subcore_barrier()

    off = cid * NP + sid * RPS
    pltpu.sync_copy(num_sh.at[pl.ds(sid * RPS, RPS)],
                    num_out.at[pl.ds(off, RPS)])
    pltpu.sync_copy(den_sh.at[pl.ds(sid * RPS, RPS)],
                    den_out.at[pl.ds(off, RPS)])


_PAD = NP - N
_BLK = 1792
_NBLK = NP // _BLK


def _row_valid(blk_idx, blk, n_valid):
    rid = blk_idx * blk + lax.broadcasted_iota(jnp.int32, (blk, 1), 0)
    return rid < n_valid


def _prologue_body(xa, xw, xr, xb, xt, wa, ww, wr, wb, wt, pb, ls, ld,
                   ha, hw, hr, hb, ht, asrc, adst):
    i = pl.program_id(0)
    valid = _row_valid(i, _BLK, N)

    def proj(x, w, b):
        h = jnp.dot(x[:], w[:].T, preferred_element_type=jnp.float32) + b
        return jnp.where(valid, h, 0.0)

    hs = {
        "arrival": proj(xa, wa, pb[0]),
        "waiting": proj(xw, ww, pb[1]),
        "resources": proj(xr, wr, pb[2]),
        "busy": proj(xb, wb, pb[3]),
        "transition": proj(xt, wt, pb[4]),
    }
    ha[:] = hs["arrival"]
    hw[:] = hs["waiting"]
    hr[:] = hs["resources"]
    hb[:] = hs["busy"]
    ht[:] = hs["transition"]
    asrc[:] = jnp.stack([(hs[_ETS[j][0]] * ls[j]).sum(-1) for j in range(8)])
    adst[:] = jnp.stack([(hs[_ETS[j][1]] * ld[j]).sum(-1) for j in range(8)])


def _prologue(x_arrival, x_waiting, x_resources, x_busy, x_transition, p1):
    pb = jnp.stack([p1["proj_b"][nt] for nt in _NTS])
    ls = jnp.stack([p1["lin_src"][k][0, 0] for k in _ET_KEYS])
    ld = jnp.stack([p1["lin_dst"][k][0, 0] for k in _ET_KEYS])
    full = lambda *shape: pl.BlockSpec(shape, lambda i: tuple(0 for _ in shape))
    xspec = lambda d: pl.BlockSpec((_BLK, d), lambda i: (i, 0))
    hspec = pl.BlockSpec((_BLK, 16), lambda i: (i, 0))
    aspec = pl.BlockSpec((8, _BLK), lambda i: (0, i))
    out_shape = ([jax.ShapeDtypeStruct((NP, 16), jnp.float32)] * 5
                 + [jax.ShapeDtypeStruct((8, NP), jnp.float32)] * 2)
    in_specs = [xspec(2), xspec(2), xspec(4), xspec(5), xspec(1),
                full(16, 2), full(16, 2), full(16, 4), full(16, 5),
                full(16, 1), full(5, 16), full(8, 16), full(8, 16)]
    out_specs = [hspec] * 5 + [aspec] * 2
    return pl.pallas_call(
        _prologue_body, grid=(_NBLK,), in_specs=in_specs,
        out_specs=out_specs, out_shape=out_shape)(
        x_arrival, x_waiting, x_resources, x_busy, x_transition,
        p1["proj_w"]["arrival"], p1["proj_w"]["waiting"],
        p1["proj_w"]["resources"], p1["proj_w"]["busy"],
        p1["proj_w"]["transition"], pb, ls, ld)


def _c1_body(n0, n1, n2, n3, n4, n5, n6, n7,
             d0, d1, d2, d3, d4, d5, d6, d7,
             kw1, kb1, w2s, b2s, ls2,
             s0, s1, s2, s3, hp0, hp1, hp2, hp3, asrc2, kxsum):
    i = pl.program_id(0)
    valid = _row_valid(i, _BLK, N)

    nums = [n0, n1, n2, n3, n4, n5, n6, n7]
    dens = [d0, d1, d2, d3, d4, d5, d6, d7]
    stks = [s0, s1, s2, s3]
    hps = [hp0, hp1, hp2, hp3]

    @pl.when(i == 0)
    def _init():
        kxsum[:] = jnp.zeros((4, 16), jnp.float32)

    acc = []
    for j in range(4):
        nm = nums[j][:]
        dn = dens[j][:]
        agg = jax.nn.relu((nm[0] + nm[1]) / (dn[0] + dn[1] + 1e-16)[:, None])
        agg = jnp.where(valid, agg, 0.0)
        stks[j][:] = agg
        kx = jnp.tanh(jnp.dot(agg, kw1[:].T,
                              preferred_element_type=jnp.float32) + kb1[:])
        kx = jnp.where(valid, kx, 0.0)
        acc.append(kx.sum(axis=0))
    kxsum[:] = kxsum[:] + jnp.stack(acc)

    a2 = []
    for j in range(4):
        nm = nums[4 + j][:]
        dn = dens[4 + j][:]
        agg = jax.nn.relu((nm[0] + nm[1]) / (dn[0] + dn[1] + 1e-16)[:, None])
        agg = jnp.where(valid, agg, 0.0)
        h2p = jnp.dot(agg, w2s[j].T, preferred_element_type=jnp.float32) + b2s[j]
        h2p = jnp.where(valid, h2p, 0.0)
        hps[j][:] = jnp.concatenate(
            [h2p, jnp.zeros((_BLK, 8), jnp.float32)], axis=1)
        a2.append((h2p * ls2[j]).sum(-1))
    asrc2[:] = jnp.stack(a2)


def _c1(nums, dens, p1, p2):
    kw1 = p1["k_lin_w"]
    kb1 = p1["k_lin_b"]
    w2s = jnp.stack([p2["proj_w"][nt] for nt in _NTS[:4]])
    b2s = jnp.stack([p2["proj_b"][nt] for nt in _NTS[:4]])
    ls2 = jnp.stack([p2["lin_src"][k][0, 0] for k in _ET_KEYS[:4]])
    grid = (_NBLK,)
    nspec = pl.BlockSpec((2, _BLK, 16), lambda i: (0, i, 0))
    dspec = pl.BlockSpec((2, _BLK), lambda i: (0, i))
    full = lambda *shape: pl.BlockSpec(shape, lambda i: tuple(0 for _ in shape))
    ospec16 = pl.BlockSpec((_BLK, 16), lambda i: (i, 0))
    out_shape = ([jax.ShapeDtypeStruct((NP, 16), jnp.float32)] * 8
                 + [jax.ShapeDtypeStruct((4, NP), jnp.float32),
                    jax.ShapeDtypeStruct((4, 16), jnp.float32)])
    out_specs = ([ospec16] * 8
                 + [pl.BlockSpec((4, _BLK), lambda i: (0, i)),
                    full(4, 16)])
    in_specs = ([nspec] * 8 + [dspec] * 8
                + [full(16, 16), full(16,), full(4, 8, 16), full(4, 8),
                   full(4, 8)])
    return pl.pallas_call(
        _c1_body, grid=grid, in_specs=in_specs, out_specs=out_specs,
        out_shape=out_shape)(*nums, *dens, kw1, kb1, w2s, b2s, ls2)


def _c2_body(s0, s1, s2, s3, kxsum, q1, w2t, b2t, ld2, hpt, adst2):
    score = (q1[:][None, :] * (kxsum[:] / N)).sum(-1)
    w = jax.nn.softmax(score)
    res = (w[0] * s0[:] + w[1] * s1[:] + w[2] * s2[:] + w[3] * s3[:])
    htr = jax.nn.relu(res)
    h2p = jnp.dot(htr, w2t[:].T, preferred_element_type=jnp.float32) + b2t[:]
    i = pl.program_id(0)
    valid = _row_valid(i, _BLK, N)
    h2p = jnp.where(valid, h2p, 0.0)
    hpt[:] = jnp.concatenate([h2p, jnp.zeros((_BLK, 8), jnp.float32)], axis=1)
    adst2[:] = jnp.stack([(h2p * ld2[j]).sum(-1) for j in range(4)])


def _c2(stks, kxsum, p1, p2):
    q1 = p1["q"]
    w2t = p2["proj_w"]["transition"]
    b2t = p2["proj_b"]["transition"]
    ld2 = jnp.stack([p2["lin_dst"][k][0, 0] for k in _ET_KEYS[:4]])
    full = lambda *shape: pl.BlockSpec(shape, lambda i: tuple(0 for _ in shape))
    bspec = pl.BlockSpec((_BLK, 16), lambda i: (i, 0))
    out_shape = [jax.ShapeDtypeStruct((NP, 16), jnp.float32),
                 jax.ShapeDtypeStruct((4, NP), jnp.float32)]
    out_specs = [bspec, pl.BlockSpec((4, _BLK), lambda i: (0, i))]
    in_specs = ([bspec] * 4 + [full(4, 16), full(16,), full(8, 16),
                               full(8,), full(4, 8)])
    return pl.pallas_call(
        _c2_body, grid=(_NBLK,), in_specs=in_specs, out_specs=out_specs,
        out_shape=out_shape)(*stks, kxsum, q1, w2t, b2t, ld2)


def _e1_body(n0, n1, n2, n3, d0, d1, d2, d3, kw2, kb2,
             a0, a1, a2, a3, kxsum):
    i = pl.program_id(0)
    valid = _row_valid(i, _BLK, N)
    nums = [n0, n1, n2, n3]
    dens = [d0, d1, d2, d3]
    aggs = [a0, a1, a2, a3]

    @pl.when(i == 0)
    def _init():
        kxsum[:] = jnp.zeros((4, 8), jnp.float32)

    acc = []
    for j in range(4):
        nm = nums[j][:]
        dn = dens[j][:]
        agg = jax.nn.relu((nm[0, :, :8] + nm[1, :, :8])
                          / (dn[0] + dn[1] + 1e-16)[:, None])
        agg = jnp.where(valid, agg, 0.0)
        aggs[j][:] = agg
        kx = jnp.tanh(jnp.dot(agg, kw2[:].T,
                              preferred_element_type=jnp.float32) + kb2[:])
        kx = jnp.where(valid, kx, 0.0)
        acc.append(kx.sum(axis=0))
    kxsum[:] = kxsum[:] + jnp.stack(acc)


def _e1(nums, dens, p2):
    grid = (_NBLK,)
    nspec = pl.BlockSpec((2, _BLK, 16), lambda i: (0, i, 0))
    dspec = pl.BlockSpec((2, _BLK), lambda i: (0, i))
    full = lambda *shape: pl.BlockSpec(shape, lambda i: tuple(0 for _ in shape))
    ospec = pl.BlockSpec((_BLK, 8), lambda i: (i, 0))
    out_shape = ([jax.ShapeDtypeStruct((NP, 8), jnp.float32)] * 4
                 + [jax.ShapeDtypeStruct((4, 8), jnp.float32)])
    out_specs = [ospec] * 4 + [full(4, 8)]
    in_specs = [nspec] * 4 + [dspec] * 4 + [full(8, 8), full(8,)]
    return pl.pallas_call(
        _e1_body, grid=grid, in_specs=in_specs, out_specs=out_specs,
        out_shape=out_shape)(*nums, *dens, p2["k_lin_w"], p2["k_lin_b"])


def _e2_body(a0, a1, a2, a3, kxsum, q2, res, colmax):
    score = (q2[:][None, :] * (kxsum[:] / N)).sum(-1)
    w = jax.nn.softmax(score)
    r = w[0] * a0[:] + w[1] * a1[:] + w[2] * a2[:] + w[3] * a3[:]
    i = pl.program_id(0)
    valid = _row_valid(i, _BLK, N)
    res[:] = r
    rm = jnp.where(valid, r, -jnp.inf).max(axis=0, keepdims=True)

    @pl.when(i == 0)
    def _init():
        colmax[:] = jnp.full((1, 8), -jnp.inf, jnp.float32)

    colmax[:] = jnp.maximum(colmax[:], rm)


def _e2(aggs, kxsum, p2):
    full = lambda *shape: pl.BlockSpec(shape, lambda i: tuple(0 for _ in shape))
    bspec = pl.BlockSpec((_BLK, 8), lambda i: (i, 0))
    out_shape = [jax.ShapeDtypeStruct((NP, 8), jnp.float32),
                 jax.ShapeDtypeStruct((1, 8), jnp.float32)]
    out_specs = [bspec, full(1, 8)]
    in_specs = [bspec] * 4 + [full(4, 8), full(8,)]
    return pl.pallas_call(
        _e2_body, grid=(_NBLK,), in_specs=in_specs, out_specs=out_specs,
        out_shape=out_shape)(*aggs, kxsum, p2["q"])


def _e3_body(res, colmax, ex, colsum):
    i = pl.program_id(0)
    valid = _row_valid(i, _BLK, N)
    e = jnp.exp(res[:] - colmax[:])
    e = jnp.where(valid, e, 0.0)
    ex[:] = e

    @pl.when(i == 0)
    def _init():
        colsum[:] = jnp.zeros((1, 8), jnp.float32)

    colsum[:] = colsum[:] + e.sum(axis=0, keepdims=True)


def _e3(res, colmax):
    full = lambda *shape: pl.BlockSpec(shape, lambda i: tuple(0 for _ in shape))
    bspec = pl.BlockSpec((_BLK, 8), lambda i: (i, 0))
    out_shape = [jax.ShapeDtypeStruct((NP, 8), jnp.float32),
                 jax.ShapeDtypeStruct((1, 8), jnp.float32)]
    return pl.pallas_call(
        _e3_body, grid=(_NBLK,), in_specs=[bspec, full(1, 8)],
        out_specs=[bspec, full(1, 8)],
        out_shape=out_shape)(res, colmax)


def _e4_body(ex, colsum, out):
    out[:] = ex[:] / colsum[:]


def _e4(ex, colsum):
    bspec = pl.BlockSpec((_BLK, 8), lambda i: (i, 0))
    full = lambda *shape: pl.BlockSpec(shape, lambda i: tuple(0 for _ in shape))
    return pl.pallas_call(
        _e4_body, grid=(_NBLK,), in_specs=[bspec, full(1, 8)],
        out_specs=bspec,
        out_shape=jax.ShapeDtypeStruct((NP, 8), jnp.float32))(ex, colsum)


def _pad_edges(ei):
    es = jnp.concatenate([ei[0].astype(jnp.int32),
                          jnp.zeros((EP - E,), jnp.int32)])
    ed = jnp.concatenate([ei[1].astype(jnp.int32),
                          jnp.full((EP - E,), NP - 1, jnp.int32)])
    return es.reshape(EROWS, 128), ed.reshape(EROWS, 128)


def kernel(x_arrival, x_waiting, x_resources, x_busy, x_transition,
           ei_a2t, ei_w2t, ei_r2t, ei_b2t, ei_t2a, ei_t2w, ei_t2r, ei_t2b,
           params1, params2):
    p1, p2 = params1, params2
    eis = [ei_a2t, ei_w2t, ei_r2t, ei_b2t, ei_t2a, ei_t2w, ei_t2r, ei_t2b]
    epads = [_pad_edges(ei) for ei in eis]

    ha, hw, hr, hb, ht, asrc1, adst1 = _prologue(
        x_arrival, x_waiting, x_resources, x_busy, x_transition, p1)
    htabs = {"arrival": ha, "waiting": hw, "resources": hr, "busy": hb,
             "transition": ht}

    nums, dens = [], []
    for j in range(8):
        es, ed = epads[j]
        num, den = _sc_edge_pass(es, ed, htabs[_ETS[j][0]],
                                 asrc1[j], adst1[j])
        nums.append(num.reshape(2, NP, 16))
        dens.append(den.reshape(2, NP))

    c1out = _c1(nums, dens, p1, p2)
    stks = c1out[0:4]
    hps = c1out[4:8]
    asrc2 = c1out[8]
    kxsum1 = c1out[9]

    hpt, adst2 = _c2(stks, kxsum1, p1, p2)

    nums2, dens2 = [], []
    for j in range(4):
        es, ed = epads[j]
        num, den = _sc_edge_pass(es, ed, hps[j], asrc2[j], adst2[j])
        nums2.append(num.reshape(2, NP, 16))
        dens2.append(den.reshape(2, NP))

    e1out = _e1(nums2, dens2, p2)
    aggs2 = e1out[0:4]
    kxsum2 = e1out[4]
    res, colmax = _e2(aggs2, kxsum2, p2)
    ex, colsum = _e3(res, colmax)
    out = _e4(ex, colsum)
    return out[:N]
